# baseline (speedup 1.0000x reference)
import functools

import jax
import jax.numpy as jnp
from jax.experimental import pallas as pl
from jax.experimental.pallas import tpu as pltpu

_TILE = 1024


def _fractal_proj_kernel(csT_ref, W_ref, scale_ref, out_ref):
    cr = csT_ref[0:1, :]
    ci = csT_ref[1:2, :]
    zr, zi = cr, ci
    vals = [zr, zi]
    for _ in range(7):
        zr, zi = zr * zr - zi * zi + cr, 2.0 * zr * zi + ci
        vals.append(zr)
        vals.append(zi)
    featsT = jnp.concatenate(vals, axis=0)
    out = jax.lax.dot_general(
        featsT, W_ref[...],
        dimension_numbers=(((0,), (1,)), ((), ())),
        preferred_element_type=jnp.float32,
    )
    out_ref[...] = out * scale_ref[0]


@jax.jit
def kernel(token_ids, cs, W, scale):
    B, L, _ = cs.shape
    D, F = W.shape
    BL = B * L
    csT = cs.reshape(BL, 2).T
    scale_arr = jnp.reshape(scale, (1,))
    grid = (BL // _TILE,)
    out = pl.pallas_call(
        _fractal_proj_kernel,
        grid=grid,
        in_specs=[
            pl.BlockSpec((2, _TILE), lambda i: (0, i)),
            pl.BlockSpec((D, F), lambda i: (0, 0)),
            pl.BlockSpec(memory_space=pltpu.SMEM),
        ],
        out_specs=pl.BlockSpec((_TILE, D), lambda i: (i, 0)),
        out_shape=jax.ShapeDtypeStruct((BL, D), jnp.float32),
        compiler_params=pltpu.CompilerParams(
            dimension_semantics=("parallel",),
        ),
    )(csT, W, scale_arr)
    return out.reshape(B, L, D)


# speedup vs baseline: 1.0347x; 1.0347x over previous
import functools

import jax
import jax.numpy as jnp
from jax.experimental import pallas as pl
from jax.experimental.pallas import tpu as pltpu

_TILE = 1024


def _fractal_proj_kernel(csT_ref, Wt_ref, scale_ref, out_ref):
    cr = csT_ref[0:1, :]
    ci = csT_ref[1:2, :]
    zr, zi = cr, ci
    vals = [zr, zi]
    for _ in range(7):
        zr, zi = zr * zr - zi * zi + cr, 2.0 * zr * zi + ci
        vals.append(zr)
        vals.append(zi)
    featsT = jnp.concatenate(vals, axis=0)
    out = jax.lax.dot_general(
        featsT, Wt_ref[...],
        dimension_numbers=(((0,), (0,)), ((), ())),
        preferred_element_type=jnp.float32,
    )
    out_ref[...] = out * scale_ref[0]


@jax.jit
def kernel(token_ids, cs, W, scale):
    B, L, _ = cs.shape
    D, F = W.shape
    BL = B * L
    csT = cs.reshape(BL, 2).T
    Wt = W.T
    scale_arr = jnp.reshape(scale, (1,))
    grid = (BL // _TILE,)
    out = pl.pallas_call(
        _fractal_proj_kernel,
        grid=grid,
        in_specs=[
            pl.BlockSpec((2, _TILE), lambda i: (0, i)),
            pl.BlockSpec((F, D), lambda i: (0, 0)),
            pl.BlockSpec(memory_space=pltpu.SMEM),
        ],
        out_specs=pl.BlockSpec((_TILE, D), lambda i: (i, 0)),
        out_shape=jax.ShapeDtypeStruct((BL, D), jnp.float32),
        compiler_params=pltpu.CompilerParams(
            dimension_semantics=("parallel",),
        ),
    )(csT, Wt, scale_arr)
    return out.reshape(B, L, D)
